# baseline (speedup 1.0000x reference)
import jax
import jax.numpy as jnp
from jax import lax
from jax.experimental import pallas as pl
from jax.experimental.pallas import tpu as pltpu
from jax.experimental.pallas import tpu_sc as plsc

B = 16384
D = 100
NS = 16
SCALE = 50.0

B_TC = 16384
B_SC = 512

NUM_CORES = 2
NUM_SUBCORES = 16
NW = NUM_CORES * NUM_SUBCORES
BPW = B_SC // NW
NB = 16
NBLK = BPW // NB
NBUF = 4

_NEG = -3.0e38

_DNUMS = lax.GatherDimensionNumbers(
    offset_dims=(), collapsed_slice_dims=(0,), start_index_map=(0,)
)


def _lane_take(v, idx):
    return lax.gather(
        v,
        idx[:, None],
        _DNUMS,
        (1,),
        mode=lax.GatherScatterMode.PROMISE_IN_BOUNDS,
    )


def _insert6(m, v):
    c = jnp.minimum(m[0], v)
    m[0] = jnp.maximum(m[0], v)
    for q in (1, 2, 3, 4):
        c, m[q] = jnp.minimum(m[q], c), jnp.maximum(m[q], c)
    m[5] = jnp.maximum(m[5], c)


def _tec_body(s_hbm, z_hbm, y_hbm, ml_hbm, out_hbm, *scratch):
    s_bufs = scratch[0:NBUF]
    z_bufs = scratch[NBUF:2 * NBUF]
    y_bufs = scratch[2 * NBUF:3 * NBUF]
    sems = scratch[3 * NBUF:4 * NBUF]
    ml_v = scratch[4 * NBUF]
    o_v = scratch[4 * NBUF + 1]

    wid = lax.axis_index("c") * NUM_SUBCORES + lax.axis_index("s")
    lane = lax.iota(jnp.int32, 16)

    pltpu.sync_copy(ml_hbm, ml_v)

    def fire(blk, k):
        base = B_TC + wid * BPW + blk * NB
        pltpu.async_copy(
            s_hbm.at[pl.ds(base * D, NB * D)], s_bufs[k], sems[k])
        pltpu.async_copy(
            z_hbm.at[pl.ds(base, NB), :], z_bufs[k], sems[k])
        pltpu.async_copy(y_hbm.at[pl.ds(base, NB)], y_bufs[k], sems[k])

    def drain(blk, k):
        base = B_TC + wid * BPW + blk * NB
        pltpu.make_async_copy(
            s_hbm.at[pl.ds(base * D, NB * D)], s_bufs[k], sems[k]).wait()
        pltpu.make_async_copy(
            z_hbm.at[pl.ds(base, NB), :], z_bufs[k], sems[k]).wait()
        pltpu.make_async_copy(
            y_hbm.at[pl.ds(base, NB)], y_bufs[k], sems[k]).wait()

    def compute_block(s_v, z_v, y_v, acc):
        def row_body(i, skp1acc):
            r0 = 2 * i
            r1 = r0 + 1
            b0 = r0 * D
            b1 = r1 * D
            neg = jnp.full((16,), _NEG, jnp.float32)
            ma = [neg] * 6
            mb = [neg] * 6
            for start, jlo in ((0, 0), (16, 0), (32, 0), (48, 0),
                               (64, 0), (80, 0), (84, 12)):
                sc0 = s_v[pl.ds(b0 + start, 16)]
                sc1 = s_v[pl.ds(b1 + start, 16)]
                for j in range(jlo, 16):
                    d = start + j
                    jj = jnp.full((16,), j, jnp.int32)
                    va = z_v[r0, pl.ds(d * 16, 16)] + _lane_take(sc0, jj)
                    vb = z_v[r1, pl.ds(d * 16, 16)] + _lane_take(sc1, jj)
                    _insert6(ma, va)
                    _insert6(mb, vb)
            ta = ma[5]
            tb = mb[5]
            for sh in (8, 4, 2, 1):
                ta = ta + _lane_take(ta, lane ^ sh)
                tb = tb + _lane_take(tb, lane ^ sh)
            skp1acc = skp1acc + jnp.where(lane == 2 * i, ta, 0.0)
            return skp1acc + jnp.where(lane == 2 * i + 1, tb, 0.0)

        skp1acc = lax.fori_loop(0, 8, row_body, jnp.zeros((16,), jnp.float32))
        y16 = y_v[...]
        margins = plsc.load_gather(ml_v, [y16])
        correct = plsc.load_gather(s_v, [lane * D + y16])
        num = jnp.maximum(
            SCALE * (margins + skp1acc * (1.0 / NS) - correct), 0.0
        )
        return acc + num

    for k in range(NBUF):
        fire(k, k)

    def ring_body(i, acc):
        blk0 = NBUF * i
        for k in range(NBUF):
            blk = blk0 + k
            drain(blk, k)
            acc = compute_block(s_bufs[k], z_bufs[k], y_bufs[k], acc)

            @pl.when(blk + NBUF < NBLK)
            def _():
                fire(blk + NBUF, k)

        return acc

    acc = lax.fori_loop(
        0, NBLK // NBUF, ring_body, jnp.zeros((16,), jnp.float32)
    )
    o_v[...] = acc
    pltpu.sync_copy(o_v, out_hbm.at[wid])


def _sc_call(sf, zf, y, m_list):
    mesh = plsc.VectorSubcoreMesh(
        core_axis_name="c", subcore_axis_name="s", num_cores=NUM_CORES
    )
    scratch = (
        [pltpu.VMEM((NB * D,), jnp.float32) for _ in range(NBUF)]
        + [pltpu.VMEM((NB, D * NS), jnp.float32) for _ in range(NBUF)]
        + [pltpu.VMEM((NB,), jnp.int32) for _ in range(NBUF)]
        + [pltpu.SemaphoreType.DMA for _ in range(NBUF)]
        + [pltpu.VMEM((D,), jnp.float32), pltpu.VMEM((16,), jnp.float32)]
    )
    return pl.kernel(
        _tec_body,
        out_type=jax.ShapeDtypeStruct((NW, 16), jnp.float32),
        mesh=mesh,
        scratch_types=scratch,
        compiler_params=pltpu.CompilerParams(needs_layout_passes=False),
    )(sf, zf, y, m_list)


R8 = 128
TC_GRID = B_TC // (8 * R8)


def _tc_body(xp_ref, sp_ref, yp_ref, ml_ref, out_ref):
    f32 = jnp.float32
    i32 = jnp.int32

    gsel = (lax.broadcasted_iota(i32, (8, 128), 1) // 16
            == lax.broadcasted_iota(i32, (8, 128), 0)).astype(f32)

    sp = sp_ref[...]
    sbc = jnp.dot(sp.reshape(R8 * D, 8), gsel,
                  preferred_element_type=f32).reshape(R8, D, 128)

    xp = xp_ref[...]
    neg = jnp.full((R8, 128), _NEG, f32)
    m = [neg] * 6
    for d in range(D):
        _insert6(m, xp[:, d, :] + sbc[:, d, :])

    skp1 = jnp.dot(m[5], gsel.T, preferred_element_type=f32) * (1.0 / NS)

    yp = yp_ref[...]
    dmask = (lax.broadcasted_iota(i32, (R8, D, 8), 1) == yp[:, None, :])
    correct = jnp.sum(jnp.where(dmask, sp, 0.0), axis=1)
    ml = ml_ref[0, :]
    margins = jnp.sum(
        jnp.where(dmask, ml[None, :, None], 0.0), axis=1)

    num = jnp.maximum(SCALE * (margins + skp1 - correct), 0.0)
    total = jnp.sum(num, keepdims=True).reshape(1, 1)

    @pl.when(pl.program_id(0) == 0)
    def _():
        out_ref[...] = jnp.zeros_like(out_ref)

    out_ref[...] += total


def _tc_call(xp, spr, ypr, ml2):
    return pl.pallas_call(
        _tc_body,
        grid=(TC_GRID,),
        in_specs=[
            pl.BlockSpec((R8, D, 128), lambda i: (i, 0, 0)),
            pl.BlockSpec((R8, D, 8), lambda i: (i, 0, 0)),
            pl.BlockSpec((R8, 8), lambda i: (i, 0)),
            pl.BlockSpec((1, D), lambda i: (0, 0)),
        ],
        out_specs=pl.BlockSpec((1, 1), lambda i: (0, 0)),
        out_shape=jax.ShapeDtypeStruct((1, 1), jnp.float32),
    )(xp, spr, ypr, ml2)


@jax.jit
def kernel(s, y, Z, m_list):
    sf = s.reshape(B * D)
    zf = Z.reshape(B, D * NS)

    nb8 = B_TC // 8
    xp = (Z[:B_TC].reshape(nb8, 8, D, NS)
          .transpose(0, 2, 1, 3).reshape(nb8, D, 128))
    spr = s[:B_TC].reshape(nb8, 8, D).transpose(0, 2, 1)
    ypr = y[:B_TC].reshape(nb8, 8)
    ml2 = m_list.reshape(1, D)

    tc_part = _tc_call(xp, spr, ypr, ml2)
    return tc_part[0, 0] * (1.0 / B)


# speedup vs baseline: 2.2849x; 2.2849x over previous
import jax
import jax.numpy as jnp
from jax import lax
from jax.experimental import pallas as pl
from jax.experimental.pallas import tpu as pltpu
from jax.experimental.pallas import tpu_sc as plsc

B = 16384
D = 100
NS = 16
SCALE = 50.0

B_SC = B

NUM_CORES = 2
NUM_SUBCORES = 16
NW = NUM_CORES * NUM_SUBCORES
BPW = B_SC // NW
NB = 16
NBLK = BPW // NB
NBUF = 4

_NEG = -3.0e38

_DNUMS = lax.GatherDimensionNumbers(
    offset_dims=(), collapsed_slice_dims=(0,), start_index_map=(0,)
)


def _lane_take(v, idx):
    return lax.gather(
        v,
        idx[:, None],
        _DNUMS,
        (1,),
        mode=lax.GatherScatterMode.PROMISE_IN_BOUNDS,
    )


def _insert6(m, v):
    c = jnp.minimum(m[0], v)
    m[0] = jnp.maximum(m[0], v)
    for q in (1, 2, 3, 4):
        c, m[q] = jnp.minimum(m[q], c), jnp.maximum(m[q], c)
    m[5] = jnp.maximum(m[5], c)


def _tec_body(s_hbm, z_hbm, y_hbm, ml_hbm, out_hbm, *scratch):
    s_bufs = scratch[0:NBUF]
    z_bufs = scratch[NBUF:2 * NBUF]
    y_bufs = scratch[2 * NBUF:3 * NBUF]
    sems = scratch[3 * NBUF:4 * NBUF]
    ml_v = scratch[4 * NBUF]
    o_v = scratch[4 * NBUF + 1]

    wid = lax.axis_index("c") * NUM_SUBCORES + lax.axis_index("s")
    lane = lax.iota(jnp.int32, 16)

    pltpu.sync_copy(ml_hbm, ml_v)

    def fire(blk, k):
        base = wid * BPW + blk * NB
        pltpu.async_copy(
            s_hbm.at[pl.ds(base * D, NB * D)], s_bufs[k], sems[k])
        pltpu.async_copy(
            z_hbm.at[pl.ds(base, NB), :], z_bufs[k], sems[k])
        pltpu.async_copy(y_hbm.at[pl.ds(base, NB)], y_bufs[k], sems[k])

    def drain(blk, k):
        base = wid * BPW + blk * NB
        pltpu.make_async_copy(
            s_hbm.at[pl.ds(base * D, NB * D)], s_bufs[k], sems[k]).wait()
        pltpu.make_async_copy(
            z_hbm.at[pl.ds(base, NB), :], z_bufs[k], sems[k]).wait()
        pltpu.make_async_copy(
            y_hbm.at[pl.ds(base, NB)], y_bufs[k], sems[k]).wait()

    def compute_block(s_v, z_v, y_v, acc):
        def row_body(i, skp1acc):
            r0 = 2 * i
            r1 = r0 + 1
            b0 = r0 * D
            b1 = r1 * D
            neg = jnp.full((16,), _NEG, jnp.float32)
            ma = [neg] * 6
            mb = [neg] * 6
            for start, jlo in ((0, 0), (16, 0), (32, 0), (48, 0),
                               (64, 0), (80, 0), (84, 12)):
                sc0 = s_v[pl.ds(b0 + start, 16)]
                sc1 = s_v[pl.ds(b1 + start, 16)]
                for j in range(jlo, 16):
                    d = start + j
                    jj = jnp.full((16,), j, jnp.int32)
                    va = z_v[r0, pl.ds(d * 16, 16)] + _lane_take(sc0, jj)
                    vb = z_v[r1, pl.ds(d * 16, 16)] + _lane_take(sc1, jj)
                    _insert6(ma, va)
                    _insert6(mb, vb)
            ta = ma[5]
            tb = mb[5]
            for sh in (8, 4, 2, 1):
                ta = ta + _lane_take(ta, lane ^ sh)
                tb = tb + _lane_take(tb, lane ^ sh)
            skp1acc = skp1acc + jnp.where(lane == 2 * i, ta, 0.0)
            return skp1acc + jnp.where(lane == 2 * i + 1, tb, 0.0)

        skp1acc = lax.fori_loop(0, 8, row_body, jnp.zeros((16,), jnp.float32))
        y16 = y_v[...]
        margins = plsc.load_gather(ml_v, [y16])
        correct = plsc.load_gather(s_v, [lane * D + y16])
        num = jnp.maximum(
            SCALE * (margins + skp1acc * (1.0 / NS) - correct), 0.0
        )
        return acc + num

    for k in range(NBUF):
        fire(k, k)

    def ring_body(i, acc):
        blk0 = NBUF * i
        for k in range(NBUF):
            blk = blk0 + k
            drain(blk, k)
            acc = compute_block(s_bufs[k], z_bufs[k], y_bufs[k], acc)

            @pl.when(blk + NBUF < NBLK)
            def _():
                fire(blk + NBUF, k)

        return acc

    acc = lax.fori_loop(
        0, NBLK // NBUF, ring_body, jnp.zeros((16,), jnp.float32)
    )
    o_v[...] = acc
    pltpu.sync_copy(o_v, out_hbm.at[wid])


def _sc_call(sf, zf, y, m_list):
    mesh = plsc.VectorSubcoreMesh(
        core_axis_name="c", subcore_axis_name="s", num_cores=NUM_CORES
    )
    scratch = (
        [pltpu.VMEM((NB * D,), jnp.float32) for _ in range(NBUF)]
        + [pltpu.VMEM((NB, D * NS), jnp.float32) for _ in range(NBUF)]
        + [pltpu.VMEM((NB,), jnp.int32) for _ in range(NBUF)]
        + [pltpu.SemaphoreType.DMA for _ in range(NBUF)]
        + [pltpu.VMEM((D,), jnp.float32), pltpu.VMEM((16,), jnp.float32)]
    )
    return pl.kernel(
        _tec_body,
        out_type=jax.ShapeDtypeStruct((NW, 16), jnp.float32),
        mesh=mesh,
        scratch_types=scratch,
        compiler_params=pltpu.CompilerParams(needs_layout_passes=False),
    )(sf, zf, y, m_list)


@jax.jit
def kernel(s, y, Z, m_list):
    sf = s.reshape(B * D)
    zf = Z.reshape(B, D * NS)
    return jnp.sum(_sc_call(sf, zf, y, m_list)) * (1.0 / B)


# speedup vs baseline: 2.2991x; 1.0062x over previous
import jax
import jax.numpy as jnp
from jax import lax
from jax.experimental import pallas as pl
from jax.experimental.pallas import tpu as pltpu
from jax.experimental.pallas import tpu_sc as plsc

B = 16384
D = 100
NS = 16
SCALE = 50.0

B_SC = B

NUM_CORES = 2
NUM_SUBCORES = 16
NW = NUM_CORES * NUM_SUBCORES
BPW = B_SC // NW
NB = 16
NBLK = BPW // NB
NBUF = 2

_NEG = -3.0e38

_DNUMS = lax.GatherDimensionNumbers(
    offset_dims=(), collapsed_slice_dims=(0,), start_index_map=(0,)
)


def _lane_take(v, idx):
    return lax.gather(
        v,
        idx[:, None],
        _DNUMS,
        (1,),
        mode=lax.GatherScatterMode.PROMISE_IN_BOUNDS,
    )


def _insert6(m, v):
    c = jnp.minimum(m[0], v)
    m[0] = jnp.maximum(m[0], v)
    for q in (1, 2, 3, 4):
        c, m[q] = jnp.minimum(m[q], c), jnp.maximum(m[q], c)
    m[5] = jnp.maximum(m[5], c)


def _tec_body(s_hbm, z_hbm, y_hbm, ml_hbm, out_hbm, *scratch):
    z_bufs = scratch[0:NBUF]
    sems = scratch[NBUF:2 * NBUF]
    s_w, y_w, ml_v, o_v, sem_s = scratch[2 * NBUF:]

    wid = lax.axis_index("c") * NUM_SUBCORES + lax.axis_index("s")
    lane = lax.iota(jnp.int32, 16)

    def fire(blk, k):
        base = wid * BPW + blk * NB
        pltpu.async_copy(
            z_hbm.at[pl.ds(base, NB), :], z_bufs[k], sems[k])

    def drain(blk, k):
        base = wid * BPW + blk * NB
        pltpu.make_async_copy(
            z_hbm.at[pl.ds(base, NB), :], z_bufs[k], sems[k]).wait()

    def compute_block(blk, z_v, acc):
        row0 = blk * NB

        def row_body(i, skp1acc):
            r0 = 2 * i
            r1 = r0 + 1
            b0 = (row0 + r0) * D
            b1 = (row0 + r1) * D
            neg = jnp.full((16,), _NEG, jnp.float32)
            ma = [neg] * 6
            mb = [neg] * 6
            for start, jlo in ((0, 0), (16, 0), (32, 0), (48, 0),
                               (64, 0), (80, 0), (84, 12)):
                sc0 = s_w[pl.ds(b0 + start, 16)]
                sc1 = s_w[pl.ds(b1 + start, 16)]
                for j in range(jlo, 16):
                    d = start + j
                    jj = jnp.full((16,), j, jnp.int32)
                    va = z_v[r0, pl.ds(d * 16, 16)] + _lane_take(sc0, jj)
                    vb = z_v[r1, pl.ds(d * 16, 16)] + _lane_take(sc1, jj)
                    _insert6(ma, va)
                    _insert6(mb, vb)
            ta = ma[5]
            tb = mb[5]
            for sh in (8, 4, 2, 1):
                ta = ta + _lane_take(ta, lane ^ sh)
                tb = tb + _lane_take(tb, lane ^ sh)
            skp1acc = skp1acc + jnp.where(lane == 2 * i, ta, 0.0)
            return skp1acc + jnp.where(lane == 2 * i + 1, tb, 0.0)

        skp1acc = lax.fori_loop(0, 8, row_body, jnp.zeros((16,), jnp.float32))
        y16 = y_w[pl.ds(row0, 16)]
        margins = plsc.load_gather(ml_v, [y16])
        correct = plsc.load_gather(s_w, [(row0 + lane) * D + y16])
        num = jnp.maximum(
            SCALE * (margins + skp1acc * (1.0 / NS) - correct), 0.0
        )
        return acc + num

    for k in range(NBUF):
        fire(k, k)

    wbase = wid * BPW
    pltpu.async_copy(s_hbm.at[pl.ds(wbase * D, BPW * D)], s_w, sem_s)
    pltpu.async_copy(y_hbm.at[pl.ds(wbase, BPW)], y_w, sem_s)
    pltpu.async_copy(ml_hbm, ml_v, sem_s)
    pltpu.make_async_copy(
        s_hbm.at[pl.ds(wbase * D, BPW * D)], s_w, sem_s).wait()
    pltpu.make_async_copy(y_hbm.at[pl.ds(wbase, BPW)], y_w, sem_s).wait()
    pltpu.make_async_copy(ml_hbm, ml_v, sem_s).wait()

    def ring_body(i, acc):
        blk0 = NBUF * i
        for k in range(NBUF):
            blk = blk0 + k
            drain(blk, k)
            acc = compute_block(blk, z_bufs[k], acc)

            @pl.when(blk + NBUF < NBLK)
            def _():
                fire(blk + NBUF, k)

        return acc

    acc = lax.fori_loop(
        0, NBLK // NBUF, ring_body, jnp.zeros((16,), jnp.float32)
    )
    o_v[...] = acc
    pltpu.sync_copy(o_v, out_hbm.at[wid])


def _sc_call(sf, zf, y, m_list):
    mesh = plsc.VectorSubcoreMesh(
        core_axis_name="c", subcore_axis_name="s", num_cores=NUM_CORES
    )
    scratch = (
        [pltpu.VMEM((NB, D * NS), jnp.float32) for _ in range(NBUF)]
        + [pltpu.SemaphoreType.DMA for _ in range(NBUF)]
        + [
            pltpu.VMEM((BPW * D,), jnp.float32),
            pltpu.VMEM((BPW,), jnp.int32),
            pltpu.VMEM((D,), jnp.float32),
            pltpu.VMEM((16,), jnp.float32),
            pltpu.SemaphoreType.DMA,
        ]
    )
    return pl.kernel(
        _tec_body,
        out_type=jax.ShapeDtypeStruct((NW, 16), jnp.float32),
        mesh=mesh,
        scratch_types=scratch,
        compiler_params=pltpu.CompilerParams(needs_layout_passes=False),
    )(sf, zf, y, m_list)


@jax.jit
def kernel(s, y, Z, m_list):
    sf = s.reshape(B * D)
    zf = Z.reshape(B, D * NS)
    return jnp.sum(_sc_call(sf, zf, y, m_list)) * (1.0 / B)
